# baseline (speedup 1.0000x reference)
import functools

import jax
import jax.numpy as jnp
from jax import lax
from jax.experimental import pallas as pl

_BETA = 0.25
_NUM_Q = 4
_N_E = 1024
_E_DIM = 64
_BATCH = 4096
_TILE = 512


def _rvq_kernel(x_ref, cb_ref, cbn_ref, xq_ref, idx_ref, loss_ref):
    residual = x_ref[...]
    xq = jnp.zeros_like(residual)
    idx_cols = []
    loss_sums = []
    for l in range(_NUM_Q):
        cb = cb_ref[l]
        cbn = cbn_ref[0, l]
        rn = jnp.sum(residual * residual, axis=1, keepdims=True)
        s = lax.dot_general(residual, cb, (((1,), (1,)), ((), ())))
        d = (rn - 2.0 * s) + cbn[None, :]
        dmin = jnp.min(d, axis=1, keepdims=True)
        iota = lax.broadcasted_iota(jnp.int32, d.shape, 1)
        idx = jnp.min(jnp.where(d == dmin, iota, _N_E), axis=1)
        oh = (iota == idx[:, None]).astype(jnp.float32)
        x_res = lax.dot_general(
            oh, cb, (((1,), (0,)), ((), ())),
            precision=lax.Precision.HIGHEST,
            preferred_element_type=jnp.float32)
        t = x_res - residual
        loss_sums.append(jnp.sum(t * t))
        st = residual + t
        residual = residual - st
        xq = xq + st
        idx_cols.append(idx)
    xq_ref[...] = xq
    idx_ref[...] = jnp.stack(idx_cols, axis=1)
    loss_ref[...] = jnp.stack(loss_sums).reshape(1, 1, _NUM_Q)


@jax.jit
def kernel(x, labels, codebooks):
    del labels
    cbn = jnp.sum(codebooks * codebooks, axis=2)
    grid = _BATCH // _TILE
    xq, idx, loss_sums = pl.pallas_call(
        _rvq_kernel,
        grid=(grid,),
        in_specs=[
            pl.BlockSpec((_TILE, _E_DIM), lambda i: (i, 0)),
            pl.BlockSpec((_NUM_Q, _N_E, _E_DIM), lambda i: (0, 0, 0)),
            pl.BlockSpec((1, _NUM_Q, _N_E), lambda i: (0, 0, 0)),
        ],
        out_specs=[
            pl.BlockSpec((_TILE, _E_DIM), lambda i: (i, 0)),
            pl.BlockSpec((_TILE, _NUM_Q), lambda i: (i, 0)),
            pl.BlockSpec((1, 1, _NUM_Q), lambda i: (i, 0, 0)),
        ],
        out_shape=[
            jax.ShapeDtypeStruct((_BATCH, _E_DIM), jnp.float32),
            jax.ShapeDtypeStruct((_BATCH, _NUM_Q), jnp.int32),
            jax.ShapeDtypeStruct((grid, 1, _NUM_Q), jnp.float32),
        ],
    )(x, codebooks, cbn[None])
    per_level = loss_sums.reshape(grid, _NUM_Q).sum(axis=0) / (_BATCH * _E_DIM)
    losses = per_level + _BETA * per_level
    mean_losses = losses.mean()
    return (xq, mean_losses, idx)


# speedup vs baseline: 1.0425x; 1.0425x over previous
import jax
import jax.numpy as jnp
from jax import lax
from jax.experimental import pallas as pl

_BETA = 0.25
_NUM_Q = 4
_N_E = 1024
_E_DIM = 64
_BATCH = 4096
_TILE = 1024


def _rvq_kernel(x_ref, cb_ref, cbn_ref, xq_ref, idx_ref, loss_ref):
    residual = x_ref[...]
    xq = jnp.zeros_like(residual)
    idx_cols = []
    loss_sums = []
    for l in range(_NUM_Q):
        cb = cb_ref[l]
        cbn = cbn_ref[0, l]
        rn = jnp.sum(residual * residual, axis=1, keepdims=True)
        s = lax.dot_general(residual, cb, (((1,), (1,)), ((), ())))
        d = (rn - 2.0 * s) + cbn[None, :]
        dmin = jnp.min(d, axis=1, keepdims=True)
        iota = lax.broadcasted_iota(jnp.int32, d.shape, 1)
        idx = jnp.min(jnp.where(d == dmin, iota, _N_E), axis=1)
        oh = (iota == idx[:, None]).astype(jnp.float32)
        x_res = lax.dot_general(
            oh, cb, (((1,), (0,)), ((), ())),
            precision=lax.Precision.HIGHEST,
            preferred_element_type=jnp.float32)
        t = x_res - residual
        loss_sums.append(jnp.sum(t * t))
        st = residual + t
        residual = residual - st
        xq = xq + st
        idx_cols.append(idx)
    xq_ref[...] = xq
    idx_ref[...] = jnp.stack(idx_cols, axis=1)
    loss_ref[...] = jnp.stack(loss_sums).reshape(1, 1, _NUM_Q)


@jax.jit
def kernel(x, labels, codebooks):
    del labels
    cbn = jnp.sum(codebooks * codebooks, axis=2)
    grid = _BATCH // _TILE
    call = pl.pallas_call(
        _rvq_kernel,
        grid=(grid,),
        in_specs=[
            pl.BlockSpec((_TILE, _E_DIM), lambda i: (i, 0)),
            pl.BlockSpec((_NUM_Q, _N_E, _E_DIM), lambda i: (0, 0, 0)),
            pl.BlockSpec((1, _NUM_Q, _N_E), lambda i: (0, 0, 0)),
        ],
        out_specs=[
            pl.BlockSpec((_TILE, _E_DIM), lambda i: (i, 0)),
            pl.BlockSpec((_TILE, _NUM_Q), lambda i: (i, 0)),
            pl.BlockSpec((1, 1, _NUM_Q), lambda i: (i, 0, 0)),
        ],
        out_shape=[
            jax.ShapeDtypeStruct((_BATCH, _E_DIM), jnp.float32),
            jax.ShapeDtypeStruct((_BATCH, _NUM_Q), jnp.int32),
            jax.ShapeDtypeStruct((grid, 1, _NUM_Q), jnp.float32),
        ],
    )
    xq, idx, loss_sums = call(x, codebooks, cbn[None])
    per_level = loss_sums.reshape(grid, _NUM_Q).sum(axis=0) / (_BATCH * _E_DIM)
    losses = per_level + _BETA * per_level
    mean_losses = losses.mean()
    return (xq, mean_losses, idx)
